# baseline (device time: 27579 ns/iter reference)
import jax
import jax.numpy as jnp
from jax import lax
from jax.experimental import pallas as pl
from jax.experimental.pallas import tpu as pltpu

N_DEV = 4
EPS = 1e-5
GLOBAL_C = 2048
NCH = 8


def _body(x_hbm, t_ref, ws_ref, wh_ref, out_hbm,
          xv, outv, acc_ref, comm_ref,
          in_sems, out_sems, send_sems, recv_sems):
    b, s, c = xv.shape
    ch = s // NCH
    my = lax.axis_index("i")
    left = (my - 1) % N_DEV
    right = (my + 1) % N_DEV

    barrier_sem = pltpu.get_barrier_semaphore()
    for nbr in [left, right]:
        pl.semaphore_signal(
            barrier_sem, inc=1,
            device_id=(nbr,), device_id_type=pl.DeviceIdType.MESH,
        )

    in_cps = []
    for k in range(NCH):
        cp = pltpu.make_async_copy(
            x_hbm.at[:, pl.ds(k * ch, ch), :],
            xv.at[:, pl.ds(k * ch, ch), :],
            in_sems.at[k],
        )
        cp.start()
        in_cps.append(cp)
    for k in range(NCH):
        in_cps[k].wait()
        xc = xv[:, pl.ds(k * ch, ch), :]
        acc_ref[0:2, pl.ds(k * ch, ch)] = jnp.sum(xc, axis=-1)
        acc_ref[2:4, pl.ds(k * ch, ch)] = jnp.sum(xc * xc, axis=-1)

    pl.semaphore_wait(barrier_sem, 2)
    comm_ref[0] = acc_ref[:]
    acc = acc_ref[:]
    for h in range(N_DEV - 1):
        rdma = pltpu.make_async_remote_copy(
            src_ref=comm_ref.at[h],
            dst_ref=comm_ref.at[h + 1],
            send_sem=send_sems.at[h],
            recv_sem=recv_sems.at[h],
            device_id=(right,),
            device_id_type=pl.DeviceIdType.MESH,
        )
        rdma.start()
        if h == 0:
            scale1 = 1.0 + jnp.dot(t_ref[:], ws_ref[:],
                                   preferred_element_type=jnp.float32)
            shift = jnp.dot(t_ref[:], wh_ref[:],
                            preferred_element_type=jnp.float32)
        rdma.wait()
        acc = acc + comm_ref[h + 1]

    mean = acc[0:2] * (1.0 / GLOBAL_C)
    var = acc[2:4] * (1.0 / GLOBAL_C) - mean * mean
    rstd = lax.rsqrt(var + EPS)
    nmr = -mean * rstd

    out_cps = [None, None]
    for k in range(NCH):
        slot = k % 2
        if out_cps[slot] is not None:
            out_cps[slot].wait()
        a_c = rstd[:, k * ch:(k + 1) * ch]
        b_c = nmr[:, k * ch:(k + 1) * ch]
        xc = xv[:, pl.ds(k * ch, ch), :]
        h_ = xc * a_c[:, :, None] + b_c[:, :, None]
        out = h_ * scale1[:, None, :] + shift[:, None, :]
        outv[slot] = out.astype(outv.dtype)
        cp = pltpu.make_async_copy(
            outv.at[slot],
            out_hbm.at[:, pl.ds(k * ch, ch), :],
            out_sems.at[slot],
        )
        cp.start()
        out_cps[slot] = cp
    out_cps[0].wait()
    out_cps[1].wait()


def kernel(x, t_emb, W_scale, W_shift):
    b, s, c = x.shape
    ch = s // NCH
    return pl.pallas_call(
        _body,
        out_shape=jax.ShapeDtypeStruct((b, s, c), jnp.bfloat16),
        in_specs=[
            pl.BlockSpec(memory_space=pl.ANY),
            pl.BlockSpec(memory_space=pltpu.VMEM),
            pl.BlockSpec(memory_space=pltpu.VMEM),
            pl.BlockSpec(memory_space=pltpu.VMEM),
        ],
        out_specs=pl.BlockSpec(memory_space=pl.ANY),
        scratch_shapes=[
            pltpu.VMEM((b, s, c), jnp.float32),
            pltpu.VMEM((2, b, ch, c), jnp.bfloat16),
            pltpu.VMEM((2 * b, s), jnp.float32),
            pltpu.VMEM((N_DEV, 2 * b, s), jnp.float32),
            pltpu.SemaphoreType.DMA((NCH,)),
            pltpu.SemaphoreType.DMA((2,)),
            pltpu.SemaphoreType.DMA((N_DEV - 1,)),
            pltpu.SemaphoreType.DMA((N_DEV - 1,)),
        ],
        compiler_params=pltpu.CompilerParams(collective_id=0),
    )(x, t_emb, W_scale, W_shift)


# device time: 25944 ns/iter; 1.0630x vs baseline; 1.0630x over previous
import jax
import jax.numpy as jnp
from jax import lax
from jax.experimental import pallas as pl
from jax.experimental.pallas import tpu as pltpu

N_DEV = 4
EPS = 1e-5
GLOBAL_C = 2048
NCH = 4


def _body(x_hbm, t_ref, ws_ref, wh_ref, out_hbm,
          xv, outv, acc_ref, comm_ref,
          in_sems, out_sems, send_sems, recv_sems):
    b, s, c = xv.shape
    ch = s // NCH
    my = lax.axis_index("i")
    left = (my - 1) % N_DEV
    right = (my + 1) % N_DEV

    barrier_sem = pltpu.get_barrier_semaphore()
    for nbr in [left, right]:
        pl.semaphore_signal(
            barrier_sem, inc=1,
            device_id=(nbr,), device_id_type=pl.DeviceIdType.MESH,
        )

    in_cps = []
    for k in range(NCH):
        cp = pltpu.make_async_copy(
            x_hbm.at[:, pl.ds(k * ch, ch), :],
            xv.at[:, pl.ds(k * ch, ch), :],
            in_sems.at[k],
        )
        cp.start()
        in_cps.append(cp)
    for k in range(NCH):
        in_cps[k].wait()
        xc = xv[:, pl.ds(k * ch, ch), :]
        acc_ref[0:2, pl.ds(k * ch, ch)] = jnp.sum(xc, axis=-1)
        acc_ref[2:4, pl.ds(k * ch, ch)] = jnp.sum(xc * xc, axis=-1)

    pl.semaphore_wait(barrier_sem, 2)
    comm_ref[0] = acc_ref[:]
    acc = acc_ref[:]
    for h in range(N_DEV - 1):
        rdma = pltpu.make_async_remote_copy(
            src_ref=comm_ref.at[h],
            dst_ref=comm_ref.at[h + 1],
            send_sem=send_sems.at[h],
            recv_sem=recv_sems.at[h],
            device_id=(right,),
            device_id_type=pl.DeviceIdType.MESH,
        )
        rdma.start()
        if h == 0:
            scale1 = 1.0 + jnp.dot(t_ref[:], ws_ref[:],
                                   preferred_element_type=jnp.float32)
            shift = jnp.dot(t_ref[:], wh_ref[:],
                            preferred_element_type=jnp.float32)
        rdma.wait()
        acc = acc + comm_ref[h + 1]

    mean = acc[0:2] * (1.0 / GLOBAL_C)
    var = acc[2:4] * (1.0 / GLOBAL_C) - mean * mean
    rstd = lax.rsqrt(var + EPS)
    nmr = (-mean * rstd).astype(jnp.bfloat16)
    rstd = rstd.astype(jnp.bfloat16)
    scale1 = scale1.astype(jnp.bfloat16)
    shift = shift.astype(jnp.bfloat16)

    out_cps = [None, None]
    for k in range(NCH):
        slot = k % 2
        if out_cps[slot] is not None:
            out_cps[slot].wait()
        a_c = rstd[:, k * ch:(k + 1) * ch]
        b_c = nmr[:, k * ch:(k + 1) * ch]
        xc = xv[:, pl.ds(k * ch, ch), :].astype(jnp.bfloat16)
        h_ = xc * a_c[:, :, None] + b_c[:, :, None]
        outv[slot] = h_ * scale1[:, None, :] + shift[:, None, :]
        cp = pltpu.make_async_copy(
            outv.at[slot],
            out_hbm.at[:, pl.ds(k * ch, ch), :],
            out_sems.at[slot],
        )
        cp.start()
        out_cps[slot] = cp
    out_cps[0].wait()
    out_cps[1].wait()


def kernel(x, t_emb, W_scale, W_shift):
    b, s, c = x.shape
    ch = s // NCH
    return pl.pallas_call(
        _body,
        out_shape=jax.ShapeDtypeStruct((b, s, c), jnp.bfloat16),
        in_specs=[
            pl.BlockSpec(memory_space=pl.ANY),
            pl.BlockSpec(memory_space=pltpu.VMEM),
            pl.BlockSpec(memory_space=pltpu.VMEM),
            pl.BlockSpec(memory_space=pltpu.VMEM),
        ],
        out_specs=pl.BlockSpec(memory_space=pl.ANY),
        scratch_shapes=[
            pltpu.VMEM((b, s, c), jnp.float32),
            pltpu.VMEM((2, b, ch, c), jnp.bfloat16),
            pltpu.VMEM((2 * b, s), jnp.float32),
            pltpu.VMEM((N_DEV, 2 * b, s), jnp.float32),
            pltpu.SemaphoreType.DMA((NCH,)),
            pltpu.SemaphoreType.DMA((2,)),
            pltpu.SemaphoreType.DMA((N_DEV - 1,)),
            pltpu.SemaphoreType.DMA((N_DEV - 1,)),
        ],
        compiler_params=pltpu.CompilerParams(collective_id=0),
    )(x, t_emb, W_scale, W_shift)


# device time: 20844 ns/iter; 1.3231x vs baseline; 1.2447x over previous
import functools

import jax
import jax.numpy as jnp
from jax import lax
from jax.experimental import pallas as pl
from jax.experimental.pallas import tpu as pltpu

N_DEV = 4
EPS = 1e-5
GLOBAL_C = 2048
NCH = 4
NHALF = 2


def _body(x_hbm, t_ref, ws_ref, wh_ref, out_hbm,
          xv, outv, acc_ref, comm_ref,
          in_sems, out_sems, send_sems, recv_sems):
    b, s, c = xv.shape
    ch = s // NCH
    hs = s // NHALF
    my = lax.axis_index("i")
    peers = [(my + d) % N_DEV for d in (1, 2, 3)]

    barrier_sem = pltpu.get_barrier_semaphore()
    for p in peers:
        pl.semaphore_signal(
            barrier_sem, inc=1,
            device_id=(p,), device_id_type=pl.DeviceIdType.MESH,
        )

    def send_half(half):
        for d in (1, 2, 3):
            rdma = pltpu.make_async_remote_copy(
                src_ref=acc_ref.at[:, pl.ds(half * hs, hs)],
                dst_ref=comm_ref.at[my, half],
                send_sem=send_sems.at[half * 3 + (d - 1)],
                recv_sem=recv_sems.at[my, half],
                device_id=((my + d) % N_DEV,),
                device_id_type=pl.DeviceIdType.MESH,
            )
            rdma.start()

    def wait_recv_half(half):
        for d in (1, 2, 3):
            src_dev = (my - d) % N_DEV
            rdma = pltpu.make_async_remote_copy(
                src_ref=acc_ref.at[:, pl.ds(half * hs, hs)],
                dst_ref=comm_ref.at[src_dev, half],
                send_sem=send_sems.at[0],
                recv_sem=recv_sems.at[src_dev, half],
                device_id=(src_dev,),
                device_id_type=pl.DeviceIdType.MESH,
            )
            rdma.wait_recv()

    in_cps = []
    for k in range(NCH):
        cp = pltpu.make_async_copy(
            x_hbm.at[:, pl.ds(k * ch, ch), :],
            xv.at[:, pl.ds(k * ch, ch), :],
            in_sems.at[k],
        )
        cp.start()
        in_cps.append(cp)
    for k in range(NCH):
        in_cps[k].wait()
        xc = xv[:, pl.ds(k * ch, ch), :]
        acc_ref[0:2, pl.ds(k * ch, ch)] = jnp.sum(xc, axis=-1)
        acc_ref[2:4, pl.ds(k * ch, ch)] = jnp.sum(xc * xc, axis=-1)
        if k == NCH // 2 - 1:
            pl.semaphore_wait(barrier_sem, 3)
            send_half(0)
    send_half(1)

    scale1 = 1.0 + jnp.dot(t_ref[:], ws_ref[:],
                           preferred_element_type=jnp.float32)
    shift = jnp.dot(t_ref[:], wh_ref[:], preferred_element_type=jnp.float32)
    scale1 = scale1.astype(jnp.bfloat16)
    shift = shift.astype(jnp.bfloat16)

    def finalize(half):
        wait_recv_half(half)
        acc = acc_ref[:, pl.ds(half * hs, hs)]
        for src in range(N_DEV):
            acc = jnp.where(src == my, acc, acc + comm_ref[src, half])
        mean = acc[0:2] * (1.0 / GLOBAL_C)
        var = acc[2:4] * (1.0 / GLOBAL_C) - mean * mean
        rstd = lax.rsqrt(var + EPS)
        return rstd.astype(jnp.bfloat16), (-mean * rstd).astype(jnp.bfloat16)

    out_cps = [None, None]
    rstd = nmr = None
    for k in range(NCH):
        if k % (NCH // NHALF) == 0:
            rstd, nmr = finalize(k // (NCH // NHALF))
        slot = k % 2
        if out_cps[slot] is not None:
            out_cps[slot].wait()
        kh = (k % (NCH // NHALF)) * ch
        a_c = rstd[:, kh:kh + ch]
        b_c = nmr[:, kh:kh + ch]
        xc = xv[:, pl.ds(k * ch, ch), :].astype(jnp.bfloat16)
        h_ = xc * a_c[:, :, None] + b_c[:, :, None]
        outv[slot] = h_ * scale1[:, None, :] + shift[:, None, :]
        cp = pltpu.make_async_copy(
            outv.at[slot],
            out_hbm.at[:, pl.ds(k * ch, ch), :],
            out_sems.at[slot],
        )
        cp.start()
        out_cps[slot] = cp
    out_cps[0].wait()
    out_cps[1].wait()

    for half in range(NHALF):
        for d in (1, 2, 3):
            rdma = pltpu.make_async_remote_copy(
                src_ref=acc_ref.at[:, pl.ds(half * hs, hs)],
                dst_ref=comm_ref.at[my, half],
                send_sem=send_sems.at[half * 3 + (d - 1)],
                recv_sem=recv_sems.at[my, half],
                device_id=((my + d) % N_DEV,),
                device_id_type=pl.DeviceIdType.MESH,
            )
            rdma.wait_send()

    @functools.partial(pl.run_scoped, sem2=pltpu.SemaphoreType.REGULAR)
    def _(sem2):
        for p in peers:
            pl.semaphore_signal(
                sem2, inc=1,
                device_id=(p,), device_id_type=pl.DeviceIdType.MESH,
            )
        pl.semaphore_wait(sem2, 3)


def kernel(x, t_emb, W_scale, W_shift):
    b, s, c = x.shape
    ch = s // NCH
    return pl.pallas_call(
        _body,
        out_shape=jax.ShapeDtypeStruct((b, s, c), jnp.bfloat16),
        in_specs=[
            pl.BlockSpec(memory_space=pl.ANY),
            pl.BlockSpec(memory_space=pltpu.VMEM),
            pl.BlockSpec(memory_space=pltpu.VMEM),
            pl.BlockSpec(memory_space=pltpu.VMEM),
        ],
        out_specs=pl.BlockSpec(memory_space=pl.ANY),
        scratch_shapes=[
            pltpu.VMEM((b, s, c), jnp.float32),
            pltpu.VMEM((2, b, ch, c), jnp.bfloat16),
            pltpu.VMEM((2 * b, s), jnp.float32),
            pltpu.VMEM((N_DEV, NHALF, 2 * b, s // NHALF), jnp.float32),
            pltpu.SemaphoreType.DMA((NCH,)),
            pltpu.SemaphoreType.DMA((2,)),
            pltpu.SemaphoreType.DMA((3 * NHALF,)),
            pltpu.SemaphoreType.DMA((N_DEV, NHALF)),
        ],
        compiler_params=pltpu.CompilerParams(collective_id=0),
    )(x, t_emb, W_scale, W_shift)


# device time: 20613 ns/iter; 1.3379x vs baseline; 1.0112x over previous
import functools

import jax
import jax.numpy as jnp
from jax import lax
from jax.experimental import pallas as pl
from jax.experimental.pallas import tpu as pltpu

N_DEV = 4
EPS = 1e-5
GLOBAL_C = 2048
NCH = 4
NHALF = 2


def _body(x_hbm, t_hbm, ws_hbm, wh_hbm, out_ref,
          xv, tv, wsv, whv, acc_ref, comm_ref,
          in_sems, w_sems, send_sems, recv_sems):
    b, s, c = xv.shape
    ch = s // NCH
    hs = s // NHALF
    my = lax.axis_index("i")
    peers = [(my + d) % N_DEV for d in (1, 2, 3)]

    barrier_sem = pltpu.get_barrier_semaphore()
    for p in peers:
        pl.semaphore_signal(
            barrier_sem, inc=1,
            device_id=(p,), device_id_type=pl.DeviceIdType.MESH,
        )

    def send_half(half):
        for d in (1, 2, 3):
            rdma = pltpu.make_async_remote_copy(
                src_ref=acc_ref.at[:, pl.ds(half * hs, hs)],
                dst_ref=comm_ref.at[my, half],
                send_sem=send_sems.at[half * 3 + (d - 1)],
                recv_sem=recv_sems.at[my, half],
                device_id=((my + d) % N_DEV,),
                device_id_type=pl.DeviceIdType.MESH,
            )
            rdma.start()

    def wait_recv_half(half):
        for d in (1, 2, 3):
            src_dev = (my - d) % N_DEV
            rdma = pltpu.make_async_remote_copy(
                src_ref=acc_ref.at[:, pl.ds(half * hs, hs)],
                dst_ref=comm_ref.at[src_dev, half],
                send_sem=send_sems.at[0],
                recv_sem=recv_sems.at[src_dev, half],
                device_id=(src_dev,),
                device_id_type=pl.DeviceIdType.MESH,
            )
            rdma.wait_recv()

    w_cps = [
        pltpu.make_async_copy(t_hbm, tv, w_sems.at[0]),
        pltpu.make_async_copy(ws_hbm, wsv, w_sems.at[1]),
        pltpu.make_async_copy(wh_hbm, whv, w_sems.at[2]),
    ]
    for cp in w_cps:
        cp.start()
    in_cps = []
    for k in range(NCH):
        cp = pltpu.make_async_copy(
            x_hbm.at[:, pl.ds(k * ch, ch), :],
            xv.at[:, pl.ds(k * ch, ch), :],
            in_sems.at[k],
        )
        cp.start()
        in_cps.append(cp)
    for k in range(NCH):
        in_cps[k].wait()
        xc = xv[:, pl.ds(k * ch, ch), :]
        acc_ref[0:2, pl.ds(k * ch, ch)] = jnp.sum(xc, axis=-1)
        acc_ref[2:4, pl.ds(k * ch, ch)] = jnp.sum(xc * xc, axis=-1)
        if k == NCH // 2 - 1:
            pl.semaphore_wait(barrier_sem, 3)
            send_half(0)
    send_half(1)

    for cp in w_cps:
        cp.wait()
    scale1 = 1.0 + jnp.dot(tv[:], wsv[:], preferred_element_type=jnp.float32)
    shift = jnp.dot(tv[:], whv[:], preferred_element_type=jnp.float32)
    scale1 = scale1.astype(jnp.bfloat16)
    shift = shift.astype(jnp.bfloat16)

    def finalize(half):
        wait_recv_half(half)
        acc = acc_ref[:, pl.ds(half * hs, hs)]
        for src in range(N_DEV):
            acc = jnp.where(src == my, acc, acc + comm_ref[src, half])
        mean = acc[0:2] * (1.0 / GLOBAL_C)
        var = acc[2:4] * (1.0 / GLOBAL_C) - mean * mean
        rstd = lax.rsqrt(var + EPS)
        return rstd.astype(jnp.bfloat16), (-mean * rstd).astype(jnp.bfloat16)

    rstd = nmr = None
    for k in range(NCH):
        if k % (NCH // NHALF) == 0:
            rstd, nmr = finalize(k // (NCH // NHALF))
        kh = (k % (NCH // NHALF)) * ch
        a_c = rstd[:, kh:kh + ch]
        b_c = nmr[:, kh:kh + ch]
        xc = xv[:, pl.ds(k * ch, ch), :].astype(jnp.bfloat16)
        h_ = xc * a_c[:, :, None] + b_c[:, :, None]
        out_ref[:, pl.ds(k * ch, ch), :] = (
            h_ * scale1[:, None, :] + shift[:, None, :]
        )

    for half in range(NHALF):
        for d in (1, 2, 3):
            rdma = pltpu.make_async_remote_copy(
                src_ref=acc_ref.at[:, pl.ds(half * hs, hs)],
                dst_ref=comm_ref.at[my, half],
                send_sem=send_sems.at[half * 3 + (d - 1)],
                recv_sem=recv_sems.at[my, half],
                device_id=((my + d) % N_DEV,),
                device_id_type=pl.DeviceIdType.MESH,
            )
            rdma.wait_send()

    @functools.partial(pl.run_scoped, sem2=pltpu.SemaphoreType.REGULAR)
    def _(sem2):
        for p in peers:
            pl.semaphore_signal(
                sem2, inc=1,
                device_id=(p,), device_id_type=pl.DeviceIdType.MESH,
            )
        pl.semaphore_wait(sem2, 3)


def kernel(x, t_emb, W_scale, W_shift):
    b, s, c = x.shape
    return pl.pallas_call(
        _body,
        out_shape=jax.ShapeDtypeStruct((b, s, c), jnp.bfloat16),
        in_specs=[
            pl.BlockSpec(memory_space=pl.ANY),
            pl.BlockSpec(memory_space=pl.ANY),
            pl.BlockSpec(memory_space=pl.ANY),
            pl.BlockSpec(memory_space=pl.ANY),
        ],
        out_specs=pl.BlockSpec(memory_space=pltpu.VMEM),
        scratch_shapes=[
            pltpu.VMEM((b, s, c), jnp.float32),
            pltpu.VMEM(t_emb.shape, jnp.float32),
            pltpu.VMEM(W_scale.shape, jnp.float32),
            pltpu.VMEM(W_shift.shape, jnp.float32),
            pltpu.VMEM((2 * b, s), jnp.float32),
            pltpu.VMEM((N_DEV, NHALF, 2 * b, s // NHALF), jnp.float32),
            pltpu.SemaphoreType.DMA((NCH,)),
            pltpu.SemaphoreType.DMA((3,)),
            pltpu.SemaphoreType.DMA((3 * NHALF,)),
            pltpu.SemaphoreType.DMA((N_DEV, NHALF)),
        ],
        compiler_params=pltpu.CompilerParams(collective_id=0),
    )(x, t_emb, W_scale, W_shift)


# device time: 19472 ns/iter; 1.4163x vs baseline; 1.0586x over previous
import functools

import jax
import jax.numpy as jnp
from jax import lax
from jax.experimental import pallas as pl
from jax.experimental.pallas import tpu as pltpu

N_DEV = 4
EPS = 1e-5
GLOBAL_C = 2048
NCH = 8
NHALF = 4


def _body(x_hbm, sc_hbm, sh_hbm, out_ref,
          xv, scv, shv, acc_ref, comm_ref,
          in_sems, w_sems, send_sems, recv_sems):
    b, s, c = xv.shape
    ch = s // NCH
    hs = s // NHALF
    my = lax.axis_index("i")
    peers = [(my + d) % N_DEV for d in (1, 2, 3)]

    barrier_sem = pltpu.get_barrier_semaphore()
    for p in peers:
        pl.semaphore_signal(
            barrier_sem, inc=1,
            device_id=(p,), device_id_type=pl.DeviceIdType.MESH,
        )

    def send_half(half):
        for d in (1, 2, 3):
            rdma = pltpu.make_async_remote_copy(
                src_ref=acc_ref.at[:, pl.ds(half * hs, hs)],
                dst_ref=comm_ref.at[my, half],
                send_sem=send_sems.at[half * 3 + (d - 1)],
                recv_sem=recv_sems.at[my, half],
                device_id=((my + d) % N_DEV,),
                device_id_type=pl.DeviceIdType.MESH,
            )
            rdma.start()

    def wait_recv_half(half):
        for d in (1, 2, 3):
            src_dev = (my - d) % N_DEV
            rdma = pltpu.make_async_remote_copy(
                src_ref=acc_ref.at[:, pl.ds(half * hs, hs)],
                dst_ref=comm_ref.at[src_dev, half],
                send_sem=send_sems.at[0],
                recv_sem=recv_sems.at[src_dev, half],
                device_id=(src_dev,),
                device_id_type=pl.DeviceIdType.MESH,
            )
            rdma.wait_recv()

    w_cps = [
        pltpu.make_async_copy(sc_hbm, scv, w_sems.at[0]),
        pltpu.make_async_copy(sh_hbm, shv, w_sems.at[1]),
    ]
    for cp in w_cps:
        cp.start()
    in_cps = []
    for k in range(NCH):
        cp = pltpu.make_async_copy(
            x_hbm.at[:, pl.ds(k * ch, ch), :],
            xv.at[:, pl.ds(k * ch, ch), :],
            in_sems.at[k],
        )
        cp.start()
        in_cps.append(cp)
    for k in range(NCH):
        in_cps[k].wait()
        xc = xv[:, pl.ds(k * ch, ch), :]
        acc_ref[0:2, pl.ds(k * ch, ch)] = jnp.sum(xc, axis=-1)
        acc_ref[2:4, pl.ds(k * ch, ch)] = jnp.sum(xc * xc, axis=-1)
        for p in range(NHALF):
            if k == (p + 1) * (NCH // NHALF) - 1:
                if p == 0:
                    pl.semaphore_wait(barrier_sem, 3)
                send_half(p)

    for cp in w_cps:
        cp.wait()
    scale1 = scv[:]
    shift = shv[:]

    def finalize(half):
        wait_recv_half(half)
        acc = acc_ref[:, pl.ds(half * hs, hs)]
        for src in range(N_DEV):
            acc = jnp.where(src == my, acc, acc + comm_ref[src, half])
        mean = acc[0:2] * (1.0 / GLOBAL_C)
        var = acc[2:4] * (1.0 / GLOBAL_C) - mean * mean
        rstd = lax.rsqrt(var + EPS)
        return rstd.astype(jnp.bfloat16), (-mean * rstd).astype(jnp.bfloat16)

    rstd = nmr = None
    for k in range(NCH):
        if k % (NCH // NHALF) == 0:
            rstd, nmr = finalize(k // (NCH // NHALF))
        kh = (k % (NCH // NHALF)) * ch
        a_c = rstd[:, kh:kh + ch]
        b_c = nmr[:, kh:kh + ch]
        xc = xv[:, pl.ds(k * ch, ch), :].astype(jnp.bfloat16)
        h_ = xc * a_c[:, :, None] + b_c[:, :, None]
        out_ref[:, pl.ds(k * ch, ch), :] = (
            h_ * scale1[:, None, :] + shift[:, None, :]
        )

    for half in range(NHALF):
        for d in (1, 2, 3):
            rdma = pltpu.make_async_remote_copy(
                src_ref=acc_ref.at[:, pl.ds(half * hs, hs)],
                dst_ref=comm_ref.at[my, half],
                send_sem=send_sems.at[half * 3 + (d - 1)],
                recv_sem=recv_sems.at[my, half],
                device_id=((my + d) % N_DEV,),
                device_id_type=pl.DeviceIdType.MESH,
            )
            rdma.wait_send()

    @functools.partial(pl.run_scoped, sem2=pltpu.SemaphoreType.REGULAR)
    def _(sem2):
        for p in peers:
            pl.semaphore_signal(
                sem2, inc=1,
                device_id=(p,), device_id_type=pl.DeviceIdType.MESH,
            )
        pl.semaphore_wait(sem2, 3)


def kernel(x, t_emb, W_scale, W_shift):
    b, s, c = x.shape
    scale1 = (1.0 + t_emb @ W_scale).astype(jnp.bfloat16)
    shift = (t_emb @ W_shift).astype(jnp.bfloat16)
    return pl.pallas_call(
        _body,
        out_shape=jax.ShapeDtypeStruct((b, s, c), jnp.bfloat16),
        in_specs=[
            pl.BlockSpec(memory_space=pl.ANY),
            pl.BlockSpec(memory_space=pl.ANY),
            pl.BlockSpec(memory_space=pl.ANY),
        ],
        out_specs=pl.BlockSpec(memory_space=pltpu.VMEM),
        scratch_shapes=[
            pltpu.VMEM((b, s, c), jnp.float32),
            pltpu.VMEM((b, c), jnp.bfloat16),
            pltpu.VMEM((b, c), jnp.bfloat16),
            pltpu.VMEM((2 * b, s), jnp.float32),
            pltpu.VMEM((N_DEV, NHALF, 2 * b, s // NHALF), jnp.float32),
            pltpu.SemaphoreType.DMA((NCH,)),
            pltpu.SemaphoreType.DMA((2,)),
            pltpu.SemaphoreType.DMA((3 * NHALF,)),
            pltpu.SemaphoreType.DMA((N_DEV, NHALF)),
        ],
        compiler_params=pltpu.CompilerParams(collective_id=0),
    )(x, scale1, shift)
